# baseline (speedup 1.0000x reference)
import jax
import jax.numpy as jnp
from jax.experimental import pallas as pl
from jax.experimental.pallas import tpu as pltpu

N_NODES = 100000
MEM_DIM = 128
MSG_DIM = 128
B_ROWS = 16384
TAIL = N_NODES - B_ROWS
C = 2048
NCH = B_ROWS // C
GSLOTS = 4
SUB = 256
TC_ROWS = 13936
NT = (TAIL + TC_ROWS - 1) // TC_ROWS


def _t_rows(j):
    return min(TC_ROWS, TAIL - j * TC_ROWS)


def _gru_chunk(h, x, wih, whh, bih, bhh):
    gi = jnp.dot(x, wih, preferred_element_type=jnp.float32) + bih
    gh = jnp.dot(h, whh, preferred_element_type=jnp.float32) + bhh
    i_r = gi[:, :MEM_DIM]
    i_z = gi[:, MEM_DIM:2 * MEM_DIM]
    i_n = gi[:, 2 * MEM_DIM:]
    h_r = gh[:, :MEM_DIM]
    h_z = gh[:, MEM_DIM:2 * MEM_DIM]
    h_n = gh[:, 2 * MEM_DIM:]
    r = jax.nn.sigmoid(i_r + h_r)
    z = jax.nn.sigmoid(i_z + h_z)
    n = jnp.tanh(i_n + r * h_n)
    return n + z * (h - n)


def _body(msg_hbm, mem_hbm, ts_hbm, lu_hbm, wih_ref, whh_ref, bih_ref, bhh_ref,
          out_mem_hbm, out_lu_hbm,
          mem_buf, msg_buf, out_buf, tbuf, lu_buf, ts_buf,
          gin_sems, gout_sems, tin_sems, tout_sems,
          lu_in_sem, lu_out_sem, ts_in_sem, ts_out_sem):

    def gin(k):
        s = k % GSLOTS
        return (pltpu.make_async_copy(mem_hbm.at[pl.ds(k * C, C), :],
                                      mem_buf.at[s], gin_sems.at[s, 0]),
                pltpu.make_async_copy(msg_hbm.at[pl.ds(k * C, C), :],
                                      msg_buf.at[s], gin_sems.at[s, 1]))

    def gout(k):
        s = k % GSLOTS
        return pltpu.make_async_copy(out_buf.at[s],
                                     out_mem_hbm.at[pl.ds(k * C, C), :],
                                     gout_sems.at[s])

    def tin(j):
        r = _t_rows(j)
        return pltpu.make_async_copy(
            mem_hbm.at[pl.ds(B_ROWS + j * TC_ROWS, r), :],
            tbuf.at[j, pl.ds(0, r)], tin_sems.at[j])

    def tout(j):
        r = _t_rows(j)
        return pltpu.make_async_copy(
            tbuf.at[j, pl.ds(0, r)],
            out_mem_hbm.at[pl.ds(B_ROWS + j * TC_ROWS, r), :], tout_sems.at[j])

    lu_in = pltpu.make_async_copy(lu_hbm.at[pl.ds(B_ROWS, TAIL)], lu_buf, lu_in_sem)
    lu_out = pltpu.make_async_copy(lu_buf, out_lu_hbm.at[pl.ds(B_ROWS, TAIL)], lu_out_sem)
    ts_in = pltpu.make_async_copy(ts_hbm, ts_buf, ts_in_sem)
    ts_out = pltpu.make_async_copy(ts_buf, out_lu_hbm.at[pl.ds(0, B_ROWS)], ts_out_sem)

    for k in range(min(2, NCH)):
        for cp in gin(k):
            cp.start()
    tin(0).start()
    tin(1).start()
    lu_in.start()
    ts_in.start()

    for k in range(NCH):
        s = k % GSLOTS
        for cp in gin(k):
            cp.wait()
        if k >= GSLOTS:
            gout(k - GSLOTS).wait()
        if k + 2 < NCH:
            for cp in gin(k + 2):
                cp.start()
        if k + 2 < NT:
            tin(k + 2).start()
        for sub in range(C // SUB):
            rs = slice(sub * SUB, (sub + 1) * SUB)
            out_buf[s, rs, :] = _gru_chunk(
                mem_buf[s, rs, :], msg_buf[s, rs, :],
                wih_ref[...], whh_ref[...], bih_ref[...], bhh_ref[...])
        gout(k).start()
        if k < NT:
            tin(k).wait()
            tout(k).start()
        if k == 0:
            ts_in.wait()
            ts_out.start()
        if k == 1:
            lu_in.wait()
            lu_out.start()

    for j in range(NCH, NT):
        if j + 2 < NT + 2 and j + 2 < NT:
            tin(j + 2).start()
        tin(j).wait()
        tout(j).start()

    for k in range(max(0, NCH - GSLOTS), NCH):
        gout(k).wait()
    for j in range(NT):
        tout(j).wait()
    lu_out.wait()
    ts_out.wait()


def kernel(unique_node_ids, unique_messages, timestamps, memory, last_update,
           W_ih, W_hh, b_ih, b_hh):
    del unique_node_ids
    wih_t = W_ih.T
    whh_t = W_hh.T
    bih = b_ih.reshape(1, -1)
    bhh = b_hh.reshape(1, -1)

    hbm = pl.BlockSpec(memory_space=pltpu.MemorySpace.HBM)
    vmem = pl.BlockSpec(memory_space=pltpu.MemorySpace.VMEM)

    updated_memory, updated_last_update = pl.pallas_call(
        _body,
        in_specs=[hbm, hbm, hbm, hbm, vmem, vmem, vmem, vmem],
        out_specs=[hbm, hbm],
        out_shape=[
            jax.ShapeDtypeStruct((N_NODES, MEM_DIM), jnp.float32),
            jax.ShapeDtypeStruct((N_NODES,), jnp.float32),
        ],
        scratch_shapes=[
            pltpu.VMEM((GSLOTS, C, MEM_DIM), jnp.float32),
            pltpu.VMEM((GSLOTS, C, MSG_DIM), jnp.float32),
            pltpu.VMEM((GSLOTS, C, MEM_DIM), jnp.float32),
            pltpu.VMEM((NT, TC_ROWS, MEM_DIM), jnp.float32),
            pltpu.VMEM((TAIL,), jnp.float32),
            pltpu.VMEM((B_ROWS,), jnp.float32),
            pltpu.SemaphoreType.DMA((GSLOTS, 2)),
            pltpu.SemaphoreType.DMA((GSLOTS,)),
            pltpu.SemaphoreType.DMA((NT,)),
            pltpu.SemaphoreType.DMA((NT,)),
            pltpu.SemaphoreType.DMA,
            pltpu.SemaphoreType.DMA,
            pltpu.SemaphoreType.DMA,
            pltpu.SemaphoreType.DMA,
        ],
    )(unique_messages, memory, timestamps, last_update, wih_t, whh_t, bih, bhh)

    return updated_memory, updated_last_update


# speedup vs baseline: 1.0117x; 1.0117x over previous
import jax
import jax.numpy as jnp
from jax.experimental import pallas as pl
from jax.experimental.pallas import tpu as pltpu

N_NODES = 100000
MEM_DIM = 128
MSG_DIM = 128
B_ROWS = 16384
TAIL = N_NODES - B_ROWS
C = 2048
NCH = B_ROWS // C
GSLOTS = 4
SUB = 256
TC_ROWS = 8192
NT = (TAIL + TC_ROWS - 1) // TC_ROWS


def _t_rows(j):
    return min(TC_ROWS, TAIL - j * TC_ROWS)


def _gru_chunk(h, x, wih, whh, bih, bhh):
    gi = jnp.dot(x, wih, preferred_element_type=jnp.float32) + bih
    gh = jnp.dot(h, whh, preferred_element_type=jnp.float32) + bhh
    i_r = gi[:, :MEM_DIM]
    i_z = gi[:, MEM_DIM:2 * MEM_DIM]
    i_n = gi[:, 2 * MEM_DIM:]
    h_r = gh[:, :MEM_DIM]
    h_z = gh[:, MEM_DIM:2 * MEM_DIM]
    h_n = gh[:, 2 * MEM_DIM:]
    r = jax.nn.sigmoid(i_r + h_r)
    z = jax.nn.sigmoid(i_z + h_z)
    n = jnp.tanh(i_n + r * h_n)
    return n + z * (h - n)


def _body(msg_hbm, mem_hbm, ts_hbm, lu_hbm, wih_ref, whh_ref, bih_ref, bhh_ref,
          out_mem_hbm, out_lu_hbm,
          mem_buf, msg_buf, out_buf, tbuf, lu_buf, ts_buf,
          gin_sems, gout_sems, tin_sems, tout_sems,
          lu_in_sem, lu_out_sem, ts_in_sem, ts_out_sem):

    def gin(k):
        s = k % GSLOTS
        return (pltpu.make_async_copy(mem_hbm.at[pl.ds(k * C, C), :],
                                      mem_buf.at[s], gin_sems.at[s, 0]),
                pltpu.make_async_copy(msg_hbm.at[pl.ds(k * C, C), :],
                                      msg_buf.at[s], gin_sems.at[s, 1]))

    def gout(k):
        s = k % GSLOTS
        return pltpu.make_async_copy(out_buf.at[s],
                                     out_mem_hbm.at[pl.ds(k * C, C), :],
                                     gout_sems.at[s])

    def tin(j):
        r = _t_rows(j)
        return pltpu.make_async_copy(
            mem_hbm.at[pl.ds(B_ROWS + j * TC_ROWS, r), :],
            tbuf.at[j, pl.ds(0, r)], tin_sems.at[j])

    def tout(j):
        r = _t_rows(j)
        return pltpu.make_async_copy(
            tbuf.at[j, pl.ds(0, r)],
            out_mem_hbm.at[pl.ds(B_ROWS + j * TC_ROWS, r), :], tout_sems.at[j])

    lu_in = pltpu.make_async_copy(lu_hbm.at[pl.ds(B_ROWS, TAIL)], lu_buf, lu_in_sem)
    lu_out = pltpu.make_async_copy(lu_buf, out_lu_hbm.at[pl.ds(B_ROWS, TAIL)], lu_out_sem)
    ts_in = pltpu.make_async_copy(ts_hbm, ts_buf, ts_in_sem)
    ts_out = pltpu.make_async_copy(ts_buf, out_lu_hbm.at[pl.ds(0, B_ROWS)], ts_out_sem)

    for k in range(min(2, NCH)):
        for cp in gin(k):
            cp.start()
    tin(0).start()
    tin(1).start()
    lu_in.start()
    ts_in.start()

    for k in range(NCH):
        s = k % GSLOTS
        for cp in gin(k):
            cp.wait()
        if k >= GSLOTS:
            gout(k - GSLOTS).wait()
        if k + 2 < NCH:
            for cp in gin(k + 2):
                cp.start()
        if k + 2 < NT:
            tin(k + 2).start()
        for sub in range(C // SUB):
            rs = slice(sub * SUB, (sub + 1) * SUB)
            out_buf[s, rs, :] = _gru_chunk(
                mem_buf[s, rs, :], msg_buf[s, rs, :],
                wih_ref[...], whh_ref[...], bih_ref[...], bhh_ref[...])
        gout(k).start()
        if k < NT:
            tin(k).wait()
            tout(k).start()
        if k == 0:
            ts_in.wait()
            ts_out.start()
        if k == 1:
            lu_in.wait()
            lu_out.start()

    for j in range(NCH, NT):
        if j + 2 < NT + 2 and j + 2 < NT:
            tin(j + 2).start()
        tin(j).wait()
        tout(j).start()

    for k in range(max(0, NCH - GSLOTS), NCH):
        gout(k).wait()
    for j in range(NT):
        tout(j).wait()
    lu_out.wait()
    ts_out.wait()


def kernel(unique_node_ids, unique_messages, timestamps, memory, last_update,
           W_ih, W_hh, b_ih, b_hh):
    del unique_node_ids
    wih_t = W_ih.T
    whh_t = W_hh.T
    bih = b_ih.reshape(1, -1)
    bhh = b_hh.reshape(1, -1)

    hbm = pl.BlockSpec(memory_space=pltpu.MemorySpace.HBM)
    vmem = pl.BlockSpec(memory_space=pltpu.MemorySpace.VMEM)

    updated_memory, updated_last_update = pl.pallas_call(
        _body,
        in_specs=[hbm, hbm, hbm, hbm, vmem, vmem, vmem, vmem],
        out_specs=[hbm, hbm],
        out_shape=[
            jax.ShapeDtypeStruct((N_NODES, MEM_DIM), jnp.float32),
            jax.ShapeDtypeStruct((N_NODES,), jnp.float32),
        ],
        scratch_shapes=[
            pltpu.VMEM((GSLOTS, C, MEM_DIM), jnp.float32),
            pltpu.VMEM((GSLOTS, C, MSG_DIM), jnp.float32),
            pltpu.VMEM((GSLOTS, C, MEM_DIM), jnp.float32),
            pltpu.VMEM((NT, TC_ROWS, MEM_DIM), jnp.float32),
            pltpu.VMEM((TAIL,), jnp.float32),
            pltpu.VMEM((B_ROWS,), jnp.float32),
            pltpu.SemaphoreType.DMA((GSLOTS, 2)),
            pltpu.SemaphoreType.DMA((GSLOTS,)),
            pltpu.SemaphoreType.DMA((NT,)),
            pltpu.SemaphoreType.DMA((NT,)),
            pltpu.SemaphoreType.DMA,
            pltpu.SemaphoreType.DMA,
            pltpu.SemaphoreType.DMA,
            pltpu.SemaphoreType.DMA,
        ],
    )(unique_messages, memory, timestamps, last_update, wih_t, whh_t, bih, bhh)

    return updated_memory, updated_last_update
